# Initial kernel scaffold; baseline (speedup 1.0000x reference)
import jax
import jax.numpy as jnp
from jax.experimental import pallas as pl


def kernel(x, emb_words, emb_pref, emb_suff, W1, b1, W2, b2):
    raise NotImplementedError("write your pallas kernel here")


# speedup vs baseline: 1.8998x; 1.8998x over previous
import jax
import jax.numpy as jnp
from jax import lax
from jax.experimental import pallas as pl
from jax.experimental.pallas import tpu as pltpu
from jax.experimental.pallas import tpu_sc as plsc

EMB = 64
WINDOW = 5
HIDDEN = 1024
LABELS = 64
BATCH = 4096
IN_DIM = EMB * WINDOW

NC, NS = 2, 16
NW = NC * NS
ROWS = BATCH * WINDOW
RPW = ROWS // NW
CHUNK = 128
NCHUNK = RPW // CHUNK

BB = 512


def _sc_gather_body(words_hbm, pref_hbm, suff_hbm, idx_hbm, out_hbm,
                    idx_v, buf_v, gsem):
    wid = lax.axis_index("s") * NC + lax.axis_index("c")
    pltpu.sync_copy(idx_hbm.at[wid], idx_v)
    tables = (words_hbm, pref_hbm, suff_hbm)
    cps = []
    for t in range(3):
        for c in range(NCHUNK):
            cps.append(pltpu.async_copy(
                tables[t].at[idx_v.at[t, c]],
                buf_v.at[t, pl.ds(c * CHUNK, CHUNK)],
                gsem))
    for cp in cps:
        cp.wait()
    for t in range(3):
        pltpu.sync_copy(buf_v.at[t], out_hbm.at[t, pl.ds(wid * RPW, RPW)])


_SC_GATHER_CACHE = []


def _sc_gather_kernel():
    if not _SC_GATHER_CACHE:
        _SC_GATHER_CACHE.append(pl.kernel(
            _sc_gather_body,
            out_type=jax.ShapeDtypeStruct((3, ROWS, EMB), jnp.float32),
            mesh=plsc.VectorSubcoreMesh(core_axis_name="c",
                                        subcore_axis_name="s",
                                        num_cores=NC, num_subcores=NS),
            scratch_types=[
                pltpu.VMEM((3, NCHUNK, CHUNK), jnp.int32),
                pltpu.VMEM((3, RPW, EMB), jnp.float32),
                pltpu.SemaphoreType.DMA,
            ],
            compiler_params=pltpu.CompilerParams(use_tc_tiling_on_sc=False),
        ))
    return _SC_GATHER_CACHE[0]


def _mlp_body(s3_ref, w1_ref, b1_ref, w2_ref, b2_ref, out_ref):
    h = s3_ref[0] + s3_ref[1] + s3_ref[2]
    z = jnp.dot(h, w1_ref[...], preferred_element_type=jnp.float32)
    z = jnp.tanh(z + b1_ref[...])
    out_ref[...] = (
        jnp.dot(z, w2_ref[...], preferred_element_type=jnp.float32)
        + b2_ref[...])


_mlp = pl.pallas_call(
    _mlp_body,
    grid=(BATCH // BB,),
    in_specs=[
        pl.BlockSpec((3, BB, IN_DIM), lambda i: (0, i, 0)),
        pl.BlockSpec((IN_DIM, HIDDEN), lambda i: (0, 0)),
        pl.BlockSpec((1, HIDDEN), lambda i: (0, 0)),
        pl.BlockSpec((HIDDEN, LABELS), lambda i: (0, 0)),
        pl.BlockSpec((1, LABELS), lambda i: (0, 0)),
    ],
    out_specs=pl.BlockSpec((BB, LABELS), lambda i: (i, 0)),
    out_shape=jax.ShapeDtypeStruct((BATCH, LABELS), jnp.float32),
)


def kernel(x, emb_words, emb_pref, emb_suff, W1, b1, W2, b2):
    idx = jnp.transpose(x, (1, 0, 2)).reshape(3, NW, NCHUNK, CHUNK)
    idx = jnp.transpose(idx, (1, 0, 2, 3))
    gathered = _sc_gather_kernel()(emb_words, emb_pref, emb_suff, idx)
    s3 = gathered.reshape(3, BATCH, IN_DIM)
    return _mlp(s3, W1, b1.reshape(1, HIDDEN), W2, b2.reshape(1, LABELS))
